# Initial kernel scaffold; baseline (speedup 1.0000x reference)
import jax
import jax.numpy as jnp
from jax.experimental import pallas as pl


def kernel(x, edge_index, W0a, b0a, W0b, b0b, eps0, W1a, b1a, W1b, b1b, W2a, b2a, W2b, b2b, eps2, gamma, beta):
    raise NotImplementedError("write your pallas kernel here")


# speedup vs baseline: 6.2791x; 6.2791x over previous
import functools

import numpy as np
import jax
import jax.numpy as jnp
from jax import lax
from jax.experimental import pallas as pl
from jax.experimental.pallas import tpu as pltpu
from jax.experimental.pallas import tpu_sc as plsc

_N = 10000
_E = 320000
_D = 128

_NC = 2
_NS = 16
_NW = _NC * _NS
_EPW = _E // _NW
_CH = 80
_NCHUNK = _EPW // _CH
_RPT = 632
_RPT_LAST = _N - 15 * _RPT


def _sc_segment_sum(h, src3, dst3, zer):
    mesh = plsc.VectorSubcoreMesh(core_axis_name="c", subcore_axis_name="s")

    def body(h_hbm, src_hbm, dst_hbm, zer_hbm, out_hbm,
             src_v, dst_v, rows_v, acc_sh, sem):
        c = lax.axis_index("c")
        s = lax.axis_index("s")
        wid = s * _NC + c

        def on_my_rows(fn):
            @pl.when(s < _NS - 1)
            def _():
                fn(pl.ds(pl.multiple_of(s * _RPT, 8), _RPT))

            @pl.when(s == _NS - 1)
            def _():
                fn(pl.ds((_NS - 1) * _RPT, _RPT_LAST))

        on_my_rows(lambda rsl: pltpu.sync_copy(zer_hbm.at[rsl], acc_sh.at[rsl]))
        plsc.subcore_barrier()
        pltpu.sync_copy(src_hbm.at[wid], src_v)
        pltpu.sync_copy(dst_hbm.at[wid], dst_v)

        def step(j, carry):
            pltpu.async_copy(h_hbm.at[src_v.at[j]], rows_v, sem).wait()
            pltpu.sync_copy(rows_v, acc_sh.at[dst_v.at[j]], add=True)
            return carry

        lax.fori_loop(0, _NCHUNK, step, 0)
        plsc.subcore_barrier()
        on_my_rows(lambda rsl: pltpu.sync_copy(acc_sh.at[rsl], out_hbm.at[c, rsl]))

    f = pl.kernel(
        body,
        out_type=jax.ShapeDtypeStruct((_NC, _N, _D), jnp.float32),
        mesh=mesh,
        scratch_types=[
            pltpu.VMEM((_NCHUNK, _CH), jnp.int32),
            pltpu.VMEM((_NCHUNK, _CH), jnp.int32),
            pltpu.VMEM((_CH, _D), jnp.float32),
            pltpu.VMEM_SHARED((_N, _D), jnp.float32),
            pltpu.SemaphoreType.DMA,
        ],
    )
    return f(h, src3, dst3, zer)


_BR = 1000


def _tc_mlp_bn(scale, h, agg2, Wa, ba, Wb, bb, gamma, beta):
    def body(sc_ref, h_ref, a0_ref, a1_ref, wa_ref, ba_ref, wb_ref, bb_ref,
             g_ref, be_ref, o_ref):
        z = sc_ref[0] * h_ref[...] + a0_ref[...] + a1_ref[...]
        z = jnp.dot(z, wa_ref[...], preferred_element_type=jnp.float32)
        z = jnp.maximum(z + ba_ref[...], 0.0)
        y = jnp.dot(z, wb_ref[...], preferred_element_type=jnp.float32)
        y = y + bb_ref[...]
        y = y * (g_ref[...] * np.float32(1.0 / np.sqrt(1.0 + 1e-5))) + be_ref[...]
        o_ref[...] = jnp.where(y >= 0.0, y, 0.01 * y)

    dout = Wb.shape[1]
    return pl.pallas_call(
        body,
        grid=(_N // _BR,),
        in_specs=[
            pl.BlockSpec(memory_space=pltpu.SMEM),
            pl.BlockSpec((_BR, _D), lambda i: (i, 0)),
            pl.BlockSpec((_BR, _D), lambda i: (i, 0)),
            pl.BlockSpec((_BR, _D), lambda i: (i, 0)),
            pl.BlockSpec((_D, _D), lambda i: (0, 0)),
            pl.BlockSpec((1, _D), lambda i: (0, 0)),
            pl.BlockSpec((_D, dout), lambda i: (0, 0)),
            pl.BlockSpec((1, dout), lambda i: (0, 0)),
            pl.BlockSpec((1, dout), lambda i: (0, 0)),
            pl.BlockSpec((1, dout), lambda i: (0, 0)),
        ],
        out_specs=pl.BlockSpec((_BR, dout), lambda i: (i, 0)),
        out_shape=jax.ShapeDtypeStruct((_N, dout), jnp.float32),
    )(scale, h, agg2[0], agg2[1], Wa, ba.reshape(1, -1), Wb,
      bb.reshape(1, -1), gamma.reshape(1, -1), beta.reshape(1, -1))


def _tc_mlp(scale, h, agg2, Wa, ba, Wb, bb):
    def body(sc_ref, h_ref, a0_ref, a1_ref, wa_ref, ba_ref, wb_ref, bb_ref,
             o_ref):
        z = sc_ref[0] * h_ref[...] + a0_ref[...] + a1_ref[...]
        z = jnp.dot(z, wa_ref[...], preferred_element_type=jnp.float32)
        z = jnp.maximum(z + ba_ref[...], 0.0)
        y = jnp.dot(z, wb_ref[...], preferred_element_type=jnp.float32)
        o_ref[...] = y + bb_ref[...]

    dout = Wb.shape[1]
    return pl.pallas_call(
        body,
        grid=(_N // _BR,),
        in_specs=[
            pl.BlockSpec(memory_space=pltpu.SMEM),
            pl.BlockSpec((_BR, _D), lambda i: (i, 0)),
            pl.BlockSpec((_BR, _D), lambda i: (i, 0)),
            pl.BlockSpec((_BR, _D), lambda i: (i, 0)),
            pl.BlockSpec((_D, _D), lambda i: (0, 0)),
            pl.BlockSpec((1, _D), lambda i: (0, 0)),
            pl.BlockSpec((_D, dout), lambda i: (0, 0)),
            pl.BlockSpec((1, dout), lambda i: (0, 0)),
        ],
        out_specs=pl.BlockSpec((_BR, dout), lambda i: (i, 0)),
        out_shape=jax.ShapeDtypeStruct((_N, dout), jnp.float32),
    )(scale, h, agg2[0], agg2[1], Wa, ba.reshape(1, -1), Wb,
      bb.reshape(1, -1))


def kernel(x, edge_index, W0a, b0a, W0b, b0b, eps0, W1a, b1a, W1b, b1b,
           W2a, b2a, W2b, b2b, eps2, gamma, beta):
    src3 = edge_index[0].reshape(_NW, _NCHUNK, _CH)
    dst3 = edge_index[1].reshape(_NW, _NCHUNK, _CH)
    zer = jnp.zeros((_N, _D), jnp.float32)

    s0 = (1.0 + eps0).reshape(1)
    s1 = jnp.ones((1,), jnp.float32)
    s2 = (1.0 + eps2).reshape(1)

    agg = _sc_segment_sum(x, src3, dst3, zer)
    h = _tc_mlp_bn(s0, x, agg, W0a, b0a, W0b, b0b, gamma, beta)
    agg = _sc_segment_sum(h, src3, dst3, zer)
    h = _tc_mlp_bn(s1, h, agg, W1a, b1a, W1b, b1b, gamma, beta)
    agg = _sc_segment_sum(h, src3, dst3, zer)
    return _tc_mlp(s2, h, agg, W2a, b2a, W2b, b2b)
